# Initial kernel scaffold; baseline (speedup 1.0000x reference)
import jax
import jax.numpy as jnp
from jax.experimental import pallas as pl


def kernel(x, h, c, W_ih, W_hh, b_ih, b_hh, fc_W, fc_b):
    raise NotImplementedError("write your pallas kernel here")


# speedup vs baseline: 2.7005x; 2.7005x over previous
import jax
import jax.numpy as jnp
from jax.experimental import pallas as pl
from jax.experimental.pallas import tpu as pltpu

_H = 256
_G = 4 * _H
_T = 24
_BB = 1024


def _decode_body(xp_ref, h_ref, c_ref, wih_ref, wt_ref, bias_ref, fcw_ref,
                 fcb_ref, out_ref):
    xg = jnp.dot(xp_ref[...], wih_ref[...],
                 preferred_element_type=jnp.float32) + bias_ref[...]
    fc = fcw_ref[...]
    fcb = fcb_ref[0]
    lane = jax.lax.broadcasted_iota(jnp.int32, (_BB, _T), 1)

    def step(t, carry):
        h_t, c_t, acc = carry
        gates = xg + jnp.dot(h_t, wt_ref[...],
                             preferred_element_type=jnp.float32)
        ti = jnp.tanh(gates[:, 0:_H])
        tf = jnp.tanh(gates[:, _H:2 * _H])
        tg = jnp.tanh(gates[:, 2 * _H:3 * _H])
        to = jnp.tanh(gates[:, 3 * _H:_G])
        c_n = 0.5 * ((tf + 1.0) * c_t + (ti + 1.0) * tg)
        h_n = 0.5 * (to + 1.0) * jnp.tanh(c_n)
        s = jnp.sum(h_n * fc, axis=1, keepdims=True) + fcb
        acc = jnp.where(lane == t, s, acc)
        return (h_n, c_n, acc)

    _, _, acc = jax.lax.fori_loop(
        0, _T, step,
        (h_ref[...], c_ref[...], jnp.zeros((_BB, _T), jnp.float32)))
    out_ref[...] = acc


def kernel(x, h, c, W_ih, W_hh, b_ih, b_hh, fc_W, fc_b):
    B = x.shape[1]
    nb = B // _BB

    scale = jnp.concatenate([
        jnp.full((2 * _H,), 0.5, jnp.float32),
        jnp.ones((_H,), jnp.float32),
        jnp.full((_H,), 0.5, jnp.float32),
    ])
    wt = W_hh.T * scale[None, :]
    bias = ((b_ih + b_hh) * scale)[None, :]
    wih = jnp.zeros((128, _G), jnp.float32).at[0].set(W_ih[:, 0] * scale)
    xp = jnp.pad(x[0], ((0, 0), (0, 127)))

    out2 = pl.pallas_call(
        _decode_body,
        out_shape=jax.ShapeDtypeStruct((B, _T), jnp.float32),
        grid=(nb,),
        in_specs=[
            pl.BlockSpec((_BB, 128), lambda i: (i, 0)),
            pl.BlockSpec((_BB, _H), lambda i: (i, 0)),
            pl.BlockSpec((_BB, _H), lambda i: (i, 0)),
            pl.BlockSpec((128, _G), lambda i: (0, 0)),
            pl.BlockSpec((_H, _G), lambda i: (0, 0)),
            pl.BlockSpec((1, _G), lambda i: (0, 0)),
            pl.BlockSpec((1, _H), lambda i: (0, 0)),
            pl.BlockSpec(memory_space=pltpu.SMEM),
        ],
        out_specs=pl.BlockSpec((_BB, _T), lambda i: (i, 0)),
        compiler_params=pltpu.CompilerParams(
            dimension_semantics=("parallel",),
            vmem_limit_bytes=48 * 1024 * 1024,
        ),
        name="lstm_decode",
    )(xp, h[0], c[0], wih, wt, bias, fc_W, fc_b)

    return out2.T.reshape(_T, B, 1)


# speedup vs baseline: 2.8859x; 1.0687x over previous
import jax
import jax.numpy as jnp
from jax.experimental import pallas as pl
from jax.experimental.pallas import tpu as pltpu
from jax.sharding import PartitionSpec as P

_H = 256
_G = 4 * _H
_T = 24
_BB = 1024


def _decode_body(xp_ref, h_ref, c_ref, wih_ref, wt_ref, bias_ref, fcw_ref,
                 fcb_ref, out_ref):
    xg = jnp.dot(xp_ref[...], wih_ref[...],
                 preferred_element_type=jnp.float32) + bias_ref[...]
    fc = fcw_ref[...]
    fcb = fcb_ref[0]
    lane = jax.lax.broadcasted_iota(jnp.int32, (_BB, _T), 1)

    def step(t, carry):
        h_t, c_t, acc = carry
        gates = xg + jnp.dot(h_t, wt_ref[...],
                             preferred_element_type=jnp.float32)
        ti = jnp.tanh(gates[:, 0:_H])
        tf = jnp.tanh(gates[:, _H:2 * _H])
        tg = jnp.tanh(gates[:, 2 * _H:3 * _H])
        to = jnp.tanh(gates[:, 3 * _H:_G])
        c_n = 0.5 * ((tf + 1.0) * c_t + (ti + 1.0) * tg)
        h_n = 0.5 * (to + 1.0) * jnp.tanh(c_n)
        s = jnp.sum(h_n * fc, axis=1, keepdims=True) + fcb
        acc = jnp.where(lane == t, s, acc)
        return (h_n, c_n, acc)

    _, _, acc = jax.lax.fori_loop(
        0, _T, step,
        (h_ref[...], c_ref[...], jnp.zeros((_BB, _T), jnp.float32)))
    out_ref[...] = acc


def _decode_shard(xp, h0, c0, wih, wt, bias, fcw, fcb):
    b_loc = xp.shape[0]
    nb = b_loc // _BB
    return pl.pallas_call(
        _decode_body,
        out_shape=jax.ShapeDtypeStruct((b_loc, _T), jnp.float32),
        grid=(nb,),
        in_specs=[
            pl.BlockSpec((_BB, 128), lambda i: (i, 0)),
            pl.BlockSpec((_BB, _H), lambda i: (i, 0)),
            pl.BlockSpec((_BB, _H), lambda i: (i, 0)),
            pl.BlockSpec((128, _G), lambda i: (0, 0)),
            pl.BlockSpec((_H, _G), lambda i: (0, 0)),
            pl.BlockSpec((1, _G), lambda i: (0, 0)),
            pl.BlockSpec((1, _H), lambda i: (0, 0)),
            pl.BlockSpec(memory_space=pltpu.SMEM),
        ],
        out_specs=pl.BlockSpec((_BB, _T), lambda i: (i, 0)),
        compiler_params=pltpu.CompilerParams(
            dimension_semantics=("parallel",),
            vmem_limit_bytes=48 * 1024 * 1024,
        ),
        name="lstm_decode",
    )(xp, h0, c0, wih, wt, bias, fcw, fcb)


def kernel(x, h, c, W_ih, W_hh, b_ih, b_hh, fc_W, fc_b):
    B = x.shape[1]

    scale = jnp.concatenate([
        jnp.full((2 * _H,), 0.5, jnp.float32),
        jnp.ones((_H,), jnp.float32),
        jnp.full((_H,), 0.5, jnp.float32),
    ])
    wt = W_hh.T * scale[None, :]
    bias = ((b_ih + b_hh) * scale)[None, :]
    wih = jnp.zeros((128, _G), jnp.float32).at[0].set(W_ih[:, 0] * scale)
    xp = jnp.pad(x[0], ((0, 0), (0, 127)))

    ndev = len(jax.devices())
    nshard = ndev if ndev > 1 and B % (ndev * _BB) == 0 else 1
    if nshard > 1:
        mesh = jax.make_mesh((nshard,), ("b",))
        in_specs = (P("b"), P("b"), P("b"), P(), P(), P(), P(), P())
        fn = jax.shard_map(
            _decode_shard, mesh=mesh, in_specs=in_specs, out_specs=P("b"),
            check_vma=False,
        )
        args = [
            jax.reshard(a, jax.NamedSharding(mesh, s))
            for a, s in zip((xp, h[0], c[0], wih, wt, bias, fc_W, fc_b),
                            in_specs)
        ]
        out2 = fn(*args)
    else:
        out2 = _decode_shard(xp, h[0], c[0], wih, wt, bias, fc_W, fc_b)
    return out2.T.reshape(_T, B, 1)


# speedup vs baseline: 3.0079x; 1.0423x over previous
import jax
import jax.numpy as jnp
from jax.experimental import pallas as pl
from jax.experimental.pallas import tpu as pltpu
from jax.sharding import PartitionSpec as P

_H = 256
_G = 4 * _H
_T = 24
_BB = 1024


def _decode_body(xp_ref, h_ref, c_ref, wih_ref, wt_ref, bias_ref, fcw_ref,
                 fcb_ref, out_ref):
    xg = jnp.dot(xp_ref[...], wih_ref[...],
                 preferred_element_type=jnp.float32) + bias_ref[...]
    fc = fcw_ref[...]
    fcb = fcb_ref[0]
    lane = jax.lax.broadcasted_iota(jnp.int32, (_BB, _T), 1)

    def step(t, carry):
        h_t, c_t, acc = carry
        gates = xg + jnp.dot(h_t, wt_ref[...],
                             preferred_element_type=jnp.float32)
        ti = jnp.tanh(gates[:, 0:_H])
        tf = jnp.tanh(gates[:, _H:2 * _H])
        tg = jnp.tanh(gates[:, 2 * _H:3 * _H])
        to = jnp.tanh(gates[:, 3 * _H:_G])
        c_n = 0.5 * ((tf + 1.0) * c_t + (ti + 1.0) * tg)
        h_n = 0.5 * (to + 1.0) * jnp.tanh(c_n)
        s = jnp.sum(h_n * fc, axis=1, keepdims=True) + fcb
        acc = jnp.where(lane == t, s, acc)
        return (h_n, c_n, acc)

    _, _, acc = jax.lax.fori_loop(
        0, _T, step,
        (h_ref[...], c_ref[...], jnp.zeros((_BB, _T), jnp.float32)))
    out_ref[...] = acc


def _decode_shard(x0, h0, c0, wih, wt, bias, fcw, fcb):
    b_loc = x0.shape[0]
    nb = b_loc // _BB
    xp = jnp.pad(x0, ((0, 0), (0, 127)))
    h0 = h0.astype(jnp.float32)
    c0 = c0.astype(jnp.float32)
    return pl.pallas_call(
        _decode_body,
        out_shape=jax.ShapeDtypeStruct((b_loc, _T), jnp.float32),
        grid=(nb,),
        in_specs=[
            pl.BlockSpec((_BB, 128), lambda i: (i, 0)),
            pl.BlockSpec((_BB, _H), lambda i: (i, 0)),
            pl.BlockSpec((_BB, _H), lambda i: (i, 0)),
            pl.BlockSpec((128, _G), lambda i: (0, 0)),
            pl.BlockSpec((_H, _G), lambda i: (0, 0)),
            pl.BlockSpec((1, _G), lambda i: (0, 0)),
            pl.BlockSpec((1, _H), lambda i: (0, 0)),
            pl.BlockSpec(memory_space=pltpu.SMEM),
        ],
        out_specs=pl.BlockSpec((_BB, _T), lambda i: (i, 0)),
        compiler_params=pltpu.CompilerParams(
            dimension_semantics=("parallel",),
            vmem_limit_bytes=48 * 1024 * 1024,
        ),
        name="lstm_decode",
    )(xp, h0, c0, wih, wt, bias, fcw, fcb)


def kernel(x, h, c, W_ih, W_hh, b_ih, b_hh, fc_W, fc_b):
    B = x.shape[1]

    scale = jnp.concatenate([
        jnp.full((2 * _H,), 0.5, jnp.float32),
        jnp.ones((_H,), jnp.float32),
        jnp.full((_H,), 0.5, jnp.float32),
    ])
    wt = W_hh.T * scale[None, :]
    bias = ((b_ih + b_hh) * scale)[None, :]
    wih = jnp.zeros((128, _G), jnp.float32).at[0].set(W_ih[:, 0] * scale)
    h0 = h[0].astype(jnp.bfloat16)
    c0 = c[0].astype(jnp.bfloat16)

    ndev = len(jax.devices())
    nshard = ndev if ndev > 1 and B % (ndev * _BB) == 0 else 1
    if nshard > 1:
        mesh = jax.make_mesh((nshard,), ("b",))
        in_specs = (P("b"), P("b"), P("b"), P(), P(), P(), P(), P())
        fn = jax.shard_map(
            _decode_shard, mesh=mesh, in_specs=in_specs, out_specs=P("b"),
            check_vma=False,
        )
        args = [
            jax.reshard(a, jax.NamedSharding(mesh, s))
            for a, s in zip((x[0], h0, c0, wih, wt, bias, fc_W, fc_b),
                            in_specs)
        ]
        out2 = fn(*args)
    else:
        out2 = _decode_shard(x[0], h0, c0, wih, wt, bias, fc_W, fc_b)
    return out2.T.reshape(_T, B, 1)


# speedup vs baseline: 3.1322x; 1.0413x over previous
import jax
import jax.numpy as jnp
from jax.experimental import pallas as pl
from jax.experimental.pallas import tpu as pltpu

_H = 256
_G4 = 4 * _H
_T = 24
_BB = 1024
_NC = 2
_CH = _BB // _NC


def _decode_body(xp_ref, h_ref, c_ref, wh_ref, wx_ref, fcw_ref, fcb_ref,
                 out_ref, h2_s, c_s, xg_s):
    h2_s[...] = h_ref[...]
    c_s[...] = c_ref[...]
    xg_s[...] = jnp.dot(xp_ref[...], wx_ref[...],
                        preferred_element_type=jnp.float32)
    fc = fcw_ref[...]
    fcb = fcb_ref[0]
    lane = jax.lax.broadcasted_iota(jnp.int32, (_CH, _T), 1)

    def one_step(h2m, c_t, t_idx, r0):
        dot = lambda w: jnp.dot(h2m, w, preferred_element_type=jnp.float32)
        g1 = dot(wh_ref[:, 0:2 * _H]) + xg_s[r0:r0 + _CH, 0:2 * _H]
        ti = jnp.tanh(g1[:, 0:_H])
        tg = jnp.tanh(g1[:, _H:2 * _H])
        p = (ti + 1.0) * tg
        g2 = dot(wh_ref[:, 2 * _H:_G4]) + xg_s[r0:r0 + _CH, 2 * _H:_G4]
        tf = jnp.tanh(g2[:, 0:_H])
        c_n = 0.5 * ((tf + 1.0) * c_t + p)
        to = jnp.tanh(g2[:, _H:2 * _H])
        h2 = (to + 1.0) * jnp.tanh(c_n)
        s = jnp.sum(h2 * fc, axis=1, keepdims=True) + fcb
        orow = out_ref[r0:r0 + _CH, :]
        out_ref[r0:r0 + _CH, :] = jnp.where(lane == t_idx, s, orow)
        return h2.astype(jnp.bfloat16), c_n

    def pair(t, _):
        hc = [None] * _NC
        for rc in range(_NC):
            r0 = rc * _CH
            hc[rc] = one_step(h2_s[r0:r0 + _CH, :], c_s[r0:r0 + _CH, :],
                              2 * t, r0)
        for rc in range(_NC):
            r0 = rc * _CH
            h2b, cb = hc[rc]
            h2c, cc = one_step(h2b, cb, 2 * t + 1, r0)
            h2_s[r0:r0 + _CH, :] = h2c
            c_s[r0:r0 + _CH, :] = cc
        return 0

    jax.lax.fori_loop(0, _T // 2, pair, 0)


def kernel(x, h, c, W_ih, W_hh, b_ih, b_hh, fc_W, fc_b):
    B = x.shape[1]
    nb = B // _BB

    perm = jnp.concatenate([
        jnp.arange(0, _H),
        jnp.arange(2 * _H, 3 * _H),
        jnp.arange(_H, 2 * _H),
        jnp.arange(3 * _H, 4 * _H),
    ])
    scale = jnp.concatenate([
        jnp.full((_H,), 0.5, jnp.float32),
        jnp.ones((_H,), jnp.float32),
        jnp.full((2 * _H,), 0.5, jnp.float32),
    ])
    wh = (W_hh.T[:, perm] * scale[None, :] * 0.5).astype(jnp.bfloat16)
    bias = (b_ih + b_hh)[perm] * scale
    bias_hi = bias.astype(jnp.bfloat16).astype(jnp.float32)
    wx = jnp.zeros((128, _G4), jnp.float32)
    wx = wx.at[0, :].set(W_ih[perm, 0] * scale)
    wx = wx.at[1, :].set(bias_hi)
    wx = wx.at[2, :].set(bias - bias_hi)
    wx = wx.astype(jnp.bfloat16)
    xp = jnp.pad(x[0], ((0, 0), (0, 127)))
    xp = xp.at[:, 1].set(1.0).at[:, 2].set(1.0).astype(jnp.bfloat16)

    out2 = pl.pallas_call(
        _decode_body,
        out_shape=jax.ShapeDtypeStruct((B, _T), jnp.float32),
        grid=(nb,),
        in_specs=[
            pl.BlockSpec((_BB, 128), lambda i: (i, 0)),
            pl.BlockSpec((_BB, _H), lambda i: (i, 0)),
            pl.BlockSpec((_BB, _H), lambda i: (i, 0)),
            pl.BlockSpec((_H, _G4), lambda i: (0, 0)),
            pl.BlockSpec((128, _G4), lambda i: (0, 0)),
            pl.BlockSpec((1, _H), lambda i: (0, 0)),
            pl.BlockSpec(memory_space=pltpu.SMEM),
        ],
        out_specs=pl.BlockSpec((_BB, _T), lambda i: (i, 0)),
        scratch_shapes=[
            pltpu.VMEM((_BB, _H), jnp.bfloat16),
            pltpu.VMEM((_BB, _H), jnp.float32),
            pltpu.VMEM((_BB, _G4), jnp.float32),
        ],
        compiler_params=pltpu.CompilerParams(
            dimension_semantics=("parallel",),
            vmem_limit_bytes=48 * 1024 * 1024,
        ),
        name="lstm_decode",
    )(xp, (2.0 * h[0]).astype(jnp.bfloat16), c[0], wh, wx,
      0.5 * fc_W, fc_b)

    return out2.T.reshape(_T, B, 1)


# speedup vs baseline: 3.5417x; 1.1307x over previous
import jax
import jax.numpy as jnp
from jax.experimental import pallas as pl
from jax.experimental.pallas import tpu as pltpu

_H = 256
_G4 = 4 * _H
_T = 24
_BB = 1024
_NC = 2
_U = 12
_CH = _BB // _NC


def _decode_body(xp_ref, h_ref, c_ref, wh_ref, wx_ref, fcw_ref, fcb_ref,
                 out_ref, xg_s, h2a_s, h2b_s, ca_s, cb_s, aca_s, acb_s):
    h2_s = (h2a_s, h2b_s)
    c_s = (ca_s, cb_s)
    ac_s = (aca_s, acb_s)
    for rc in range(_NC):
        r0 = rc * _CH
        h2_s[rc][...] = h_ref[r0:r0 + _CH, :]
        c_s[rc][...] = c_ref[r0:r0 + _CH, :]
    xg_s[...] = jnp.dot(xp_ref[...], wx_ref[...],
                        preferred_element_type=jnp.float32)
    fc = fcw_ref[...]
    fcb = fcb_ref[0]
    lane = jax.lax.broadcasted_iota(jnp.int32, (_CH, _T), 1)

    def one_step(h2m, c_t, t_idx, rc):
        r0 = rc * _CH
        dot = lambda w: jnp.dot(h2m, w, preferred_element_type=jnp.float32)
        g1 = dot(wh_ref[:, 0:2 * _H]) + xg_s[r0:r0 + _CH, 0:2 * _H]
        ti = jnp.tanh(g1[:, 0:_H])
        tg = jnp.tanh(g1[:, _H:2 * _H])
        p = (ti + 1.0) * tg
        g2 = dot(wh_ref[:, 2 * _H:_G4]) + xg_s[r0:r0 + _CH, 2 * _H:_G4]
        tf = jnp.tanh(g2[:, 0:_H])
        c_n = 0.5 * ((tf + 1.0) * c_t + p)
        to = jnp.tanh(g2[:, _H:2 * _H])
        h2 = (to + 1.0) * jnp.tanh(c_n)
        s = jnp.sum(h2 * fc, axis=1, keepdims=True) + fcb
        ac_s[rc][...] = jnp.where(lane == t_idx, s, ac_s[rc][...])
        return h2.astype(jnp.bfloat16), c_n

    def multi(t, _):
        hs = [h2_s[rc][...] for rc in range(_NC)]
        cs = [c_s[rc][...] for rc in range(_NC)]
        for u in range(_U):
            for rc in range(_NC):
                hs[rc], cs[rc] = one_step(hs[rc], cs[rc], _U * t + u, rc)
        for rc in range(_NC):
            h2_s[rc][...] = hs[rc]
            c_s[rc][...] = cs[rc]
        return 0

    jax.lax.fori_loop(0, _T // _U, multi, 0)
    for rc in range(_NC):
        r0 = rc * _CH
        out_ref[r0:r0 + _CH, :] = ac_s[rc][...]


def kernel(x, h, c, W_ih, W_hh, b_ih, b_hh, fc_W, fc_b):
    B = x.shape[1]
    nb = B // _BB

    perm = jnp.concatenate([
        jnp.arange(0, _H),
        jnp.arange(2 * _H, 3 * _H),
        jnp.arange(_H, 2 * _H),
        jnp.arange(3 * _H, 4 * _H),
    ])
    scale = jnp.concatenate([
        jnp.full((_H,), 0.5, jnp.float32),
        jnp.ones((_H,), jnp.float32),
        jnp.full((2 * _H,), 0.5, jnp.float32),
    ])
    wh = (W_hh.T[:, perm] * scale[None, :] * 0.5).astype(jnp.bfloat16)
    bias = (b_ih + b_hh)[perm] * scale
    bias_hi = bias.astype(jnp.bfloat16).astype(jnp.float32)
    wx = jnp.zeros((128, _G4), jnp.float32)
    wx = wx.at[0, :].set(W_ih[perm, 0] * scale)
    wx = wx.at[1, :].set(bias_hi)
    wx = wx.at[2, :].set(bias - bias_hi)
    wx = wx.astype(jnp.bfloat16)
    xp = jnp.pad(x[0], ((0, 0), (0, 127)))
    xp = xp.at[:, 1].set(1.0).at[:, 2].set(1.0).astype(jnp.bfloat16)

    out2 = pl.pallas_call(
        _decode_body,
        out_shape=jax.ShapeDtypeStruct((B, _T), jnp.float32),
        grid=(nb,),
        in_specs=[
            pl.BlockSpec((_BB, 128), lambda i: (i, 0)),
            pl.BlockSpec((_BB, _H), lambda i: (i, 0)),
            pl.BlockSpec((_BB, _H), lambda i: (i, 0)),
            pl.BlockSpec((_H, _G4), lambda i: (0, 0)),
            pl.BlockSpec((128, _G4), lambda i: (0, 0)),
            pl.BlockSpec((1, _H), lambda i: (0, 0)),
            pl.BlockSpec(memory_space=pltpu.SMEM),
        ],
        out_specs=pl.BlockSpec((_BB, _T), lambda i: (i, 0)),
        scratch_shapes=[
            pltpu.VMEM((_BB, _G4), jnp.float32),
            pltpu.VMEM((_CH, _H), jnp.bfloat16),
            pltpu.VMEM((_CH, _H), jnp.bfloat16),
            pltpu.VMEM((_CH, _H), jnp.float32),
            pltpu.VMEM((_CH, _H), jnp.float32),
            pltpu.VMEM((_CH, _T), jnp.float32),
            pltpu.VMEM((_CH, _T), jnp.float32),
        ],
        compiler_params=pltpu.CompilerParams(
            dimension_semantics=("parallel",),
            vmem_limit_bytes=48 * 1024 * 1024,
        ),
        name="lstm_decode",
    )(xp, (2.0 * h[0]).astype(jnp.bfloat16), c[0], wh, wx,
      0.5 * fc_W, fc_b)

    return out2.T.reshape(_T, B, 1)


# speedup vs baseline: 3.6312x; 1.0253x over previous
import jax
import jax.numpy as jnp
from jax.experimental import pallas as pl
from jax.experimental.pallas import tpu as pltpu

_H = 256
_G4 = 4 * _H
_T = 24
_BB = 2048
_NC = 2
_U = 12
_CH = _BB // _NC


def _decode_body(xp_ref, h_ref, c_ref, wh_ref, wx_ref, fcw_ref, fcb_ref,
                 out_ref, xg_s, h2a_s, h2b_s, ca_s, cb_s, aca_s, acb_s):
    h2_s = (h2a_s, h2b_s)
    c_s = (ca_s, cb_s)
    ac_s = (aca_s, acb_s)
    for rc in range(_NC):
        r0 = rc * _CH
        h2_s[rc][...] = h_ref[r0:r0 + _CH, :]
        c_s[rc][...] = c_ref[r0:r0 + _CH, :]
    xg_s[...] = jnp.dot(xp_ref[...], wx_ref[...],
                        preferred_element_type=jnp.float32)
    fc = fcw_ref[...]
    fcb = fcb_ref[0]
    lane = jax.lax.broadcasted_iota(jnp.int32, (_CH, _T), 1)

    def one_step(h2m, c_t, t_idx, rc):
        r0 = rc * _CH
        dot = lambda w: jnp.dot(h2m, w, preferred_element_type=jnp.float32)
        g1 = dot(wh_ref[:, 0:2 * _H]) + xg_s[r0:r0 + _CH, 0:2 * _H]
        ti = jnp.tanh(g1[:, 0:_H])
        tg = jnp.tanh(g1[:, _H:2 * _H])
        p = (ti + 1.0) * tg
        g2 = dot(wh_ref[:, 2 * _H:_G4]) + xg_s[r0:r0 + _CH, 2 * _H:_G4]
        tf = jnp.tanh(g2[:, 0:_H])
        c_n = 0.5 * ((tf + 1.0) * c_t + p)
        to = jnp.tanh(g2[:, _H:2 * _H])
        h2 = (to + 1.0) * jnp.tanh(c_n)
        s = jnp.sum(h2 * fc, axis=1, keepdims=True) + fcb
        ac_s[rc][...] = jnp.where(lane == t_idx, s, ac_s[rc][...])
        return h2.astype(jnp.bfloat16), c_n

    def multi(t, _):
        hs = [h2_s[rc][...] for rc in range(_NC)]
        cs = [c_s[rc][...] for rc in range(_NC)]
        for u in range(_U):
            for rc in range(_NC):
                hs[rc], cs[rc] = one_step(hs[rc], cs[rc], _U * t + u, rc)
        for rc in range(_NC):
            h2_s[rc][...] = hs[rc]
            c_s[rc][...] = cs[rc]
        return 0

    jax.lax.fori_loop(0, _T // _U, multi, 0)
    for rc in range(_NC):
        r0 = rc * _CH
        out_ref[r0:r0 + _CH, :] = ac_s[rc][...]


def kernel(x, h, c, W_ih, W_hh, b_ih, b_hh, fc_W, fc_b):
    B = x.shape[1]
    nb = B // _BB

    perm = jnp.concatenate([
        jnp.arange(0, _H),
        jnp.arange(2 * _H, 3 * _H),
        jnp.arange(_H, 2 * _H),
        jnp.arange(3 * _H, 4 * _H),
    ])
    scale = jnp.concatenate([
        jnp.full((_H,), 0.5, jnp.float32),
        jnp.ones((_H,), jnp.float32),
        jnp.full((2 * _H,), 0.5, jnp.float32),
    ])
    wh = (W_hh.T[:, perm] * scale[None, :] * 0.5).astype(jnp.bfloat16)
    bias = (b_ih + b_hh)[perm] * scale
    bias_hi = bias.astype(jnp.bfloat16).astype(jnp.float32)
    wx = jnp.zeros((128, _G4), jnp.float32)
    wx = wx.at[0, :].set(W_ih[perm, 0] * scale)
    wx = wx.at[1, :].set(bias_hi)
    wx = wx.at[2, :].set(bias - bias_hi)
    wx = wx.astype(jnp.bfloat16)
    xp = jnp.pad(x[0], ((0, 0), (0, 127)))
    xp = xp.at[:, 1].set(1.0).at[:, 2].set(1.0).astype(jnp.bfloat16)

    out2 = pl.pallas_call(
        _decode_body,
        out_shape=jax.ShapeDtypeStruct((B, _T), jnp.float32),
        grid=(nb,),
        in_specs=[
            pl.BlockSpec((_BB, 128), lambda i: (i, 0)),
            pl.BlockSpec((_BB, _H), lambda i: (i, 0)),
            pl.BlockSpec((_BB, _H), lambda i: (i, 0)),
            pl.BlockSpec((_H, _G4), lambda i: (0, 0)),
            pl.BlockSpec((128, _G4), lambda i: (0, 0)),
            pl.BlockSpec((1, _H), lambda i: (0, 0)),
            pl.BlockSpec(memory_space=pltpu.SMEM),
        ],
        out_specs=pl.BlockSpec((_BB, _T), lambda i: (i, 0)),
        scratch_shapes=[
            pltpu.VMEM((_BB, _G4), jnp.float32),
            pltpu.VMEM((_CH, _H), jnp.bfloat16),
            pltpu.VMEM((_CH, _H), jnp.bfloat16),
            pltpu.VMEM((_CH, _H), jnp.float32),
            pltpu.VMEM((_CH, _H), jnp.float32),
            pltpu.VMEM((_CH, _T), jnp.float32),
            pltpu.VMEM((_CH, _T), jnp.float32),
        ],
        compiler_params=pltpu.CompilerParams(
            dimension_semantics=("parallel",),
            vmem_limit_bytes=48 * 1024 * 1024,
        ),
        name="lstm_decode",
    )(xp, (2.0 * h[0]).astype(jnp.bfloat16), c[0], wh, wx,
      0.5 * fc_W, fc_b)

    return out2.T.reshape(_T, B, 1)
